# baseline (speedup 1.0000x reference)
import jax
import jax.numpy as jnp
from jax import lax
from jax.experimental import pallas as pl
from jax.experimental.pallas import tpu as pltpu
from jax.experimental.pallas import tpu_sc as plsc

PATH_LEN = 20
EMBED_SIZE = 64
LANES = 16
BLK = 128


def _sc_body(ce_hbm, idx_hbm, bm_hbm, matT_hbm, out_hbm,
             idx_v, ce_v, bm_v, blk_v, out_v, sem0, sem1):
    cid = lax.axis_index("c")
    sid = lax.axis_index("s")

    @pl.when(jnp.logical_and(cid == 0, sid == 0))
    def _():
        pltpu.sync_copy(idx_hbm, idx_v.at[pl.ds(0, PATH_LEN)])
        pltpu.sync_copy(ce_hbm, ce_v)
        pltpu.sync_copy(bm_hbm, bm_v.at[pl.ds(0, PATH_LEN)])

        lane = lax.iota(jnp.int32, LANES)
        cev = [ce_v[pl.ds(c * LANES, LANES)] for c in range(EMBED_SIZE // LANES)]

        def base_of(p):
            s = plsc.load_gather(idx_v, [jnp.full((LANES,), p, jnp.int32)])[0]
            return pl.multiple_of(s - (s & jnp.int32(BLK - 1)), BLK), s & jnp.int32(BLK - 1)

        def copy_of(p, slot, sem):
            base, _ = base_of(p)
            return pltpu.make_async_copy(
                matT_hbm.at[:, pl.ds(base, BLK)], blk_v.at[slot], sem
            )

        copy_of(0, 0, sem0).start()

        def body(p, carry):
            z0, z1 = carry
            parity = p & 1

            @pl.when(p + 1 < PATH_LEN)
            def _():
                @pl.when(parity == 0)
                def _():
                    copy_of(p + 1, 1, sem1).start()

                @pl.when(parity == 1)
                def _():
                    copy_of(p + 1, 0, sem0).start()

            _, cc = base_of(p)
            ccv = jnp.full((LANES,), cc, jnp.int32)

            def consume(slot, sem):
                copy_of(p, slot, sem).wait()
                acc = plsc.load_gather(blk_v.at[slot], [lane, ccv]) * cev[0]
                for c in range(1, EMBED_SIZE // LANES):
                    rows = lane + (c * LANES)
                    acc = acc + plsc.load_gather(blk_v.at[slot], [rows, ccv]) * cev[c]
                return acc

            acc = lax.cond(
                parity == 0,
                lambda: consume(0, sem0),
                lambda: consume(1, sem1),
            )
            dot = jnp.sum(acc)
            dv = jnp.full((LANES,), dot, jnp.float32)
            z0 = jnp.where(lane == p, dv, z0)
            z1 = jnp.where(lane == (p - LANES), dv, z1)
            return z0, z1

        z0 = jnp.zeros((LANES,), jnp.float32)
        z1 = jnp.zeros((LANES,), jnp.float32)
        z0, z1 = lax.fori_loop(0, PATH_LEN, body, (z0, z1))

        z0 = z0 * bm_v[pl.ds(0, LANES)]
        z1 = z1 * bm_v[pl.ds(LANES, LANES)]
        p0 = 1.0 / (1.0 + jnp.exp(-z0))
        p1 = 1.0 / (1.0 + jnp.exp(-z1))
        p1 = jnp.where(lane < (PATH_LEN - LANES), p1, jnp.float32(1.0))
        pv = p0 * p1

        r = pv[0]
        for l in range(1, LANES):
            r = r * pv[l]
        out_v[...] = jnp.full((LANES,), r, jnp.float32)
        pltpu.sync_copy(out_v, out_hbm)


@jax.jit
def _run(ce, idx, bm, matT):
    mesh = plsc.VectorSubcoreMesh(core_axis_name="c", subcore_axis_name="s")
    f = pl.kernel(
        _sc_body,
        out_type=jax.ShapeDtypeStruct((LANES,), jnp.float32),
        mesh=mesh,
        compiler_params=pltpu.CompilerParams(needs_layout_passes=False),
        scratch_types=[
            pltpu.VMEM((2 * LANES,), jnp.int32),
            pltpu.VMEM((EMBED_SIZE,), jnp.float32),
            pltpu.VMEM((2 * LANES,), jnp.float32),
            pltpu.VMEM((2, EMBED_SIZE, BLK), jnp.float32),
            pltpu.VMEM((LANES,), jnp.float32),
            pltpu.SemaphoreType.DMA,
            pltpu.SemaphoreType.DMA,
        ],
    )
    out = f(ce, idx, bm, matT)
    return out[0]


def kernel(context_embedding, input_path_idxs, binary_multiplier, matrix):
    ce = context_embedding.reshape(EMBED_SIZE)
    idx = input_path_idxs.astype(jnp.int32)
    bm = binary_multiplier.reshape(PATH_LEN)
    return _run(ce, idx, bm, matrix.T)


# speedup vs baseline: 1.1888x; 1.1888x over previous
import jax
import jax.numpy as jnp
from jax import lax
from jax.experimental import pallas as pl
from jax.experimental.pallas import tpu as pltpu
from jax.experimental.pallas import tpu_sc as plsc

PATH_LEN = 20
EMBED_SIZE = 64
LANES = 16
BLK = 128
NSLOTS = 10


def _sc_body(ce_hbm, idx_hbm, bm_hbm, matT_hbm, out_hbm,
             idx_v, ce_v, bm_v, blk_v, out_v, sem):
    cid = lax.axis_index("c")
    sid = lax.axis_index("s")

    @pl.when(jnp.logical_and(cid == 0, sid == 0))
    def _():
        pltpu.sync_copy(idx_hbm, idx_v.at[pl.ds(0, PATH_LEN)])
        pltpu.sync_copy(ce_hbm, ce_v)
        pltpu.sync_copy(bm_hbm, bm_v.at[pl.ds(0, PATH_LEN)])

        iv1 = idx_v[pl.ds(0, LANES)]
        iv2 = idx_v[pl.ds(LANES, LANES)]
        ib1 = iv1 - (iv1 & jnp.int32(BLK - 1))
        ib2 = iv2 - (iv2 & jnp.int32(BLK - 1))
        ic1 = iv1 & jnp.int32(BLK - 1)
        ic2 = iv2 & jnp.int32(BLK - 1)

        def base_of(p):
            b = ib1[p] if p < LANES else ib2[p - LANES]
            return pl.multiple_of(b, BLK)

        def col_of(p):
            return ic1[p] if p < LANES else ic2[p - LANES]

        def fire(p):
            return pltpu.async_copy(
                matT_hbm.at[:, pl.ds(base_of(p), BLK)],
                blk_v.at[p % NSLOTS],
                sem,
            )

        lane = lax.iota(jnp.int32, LANES)
        cev = [ce_v[pl.ds(c * LANES, LANES)] for c in range(EMBED_SIZE // LANES)]

        def compute(p, z0, z1):
            cc = jnp.full((LANES,), col_of(p), jnp.int32)
            acc = plsc.load_gather(blk_v.at[p % NSLOTS], [lane, cc]) * cev[0]
            for c in range(1, EMBED_SIZE // LANES):
                rows = lane + (c * LANES)
                acc = acc + plsc.load_gather(blk_v.at[p % NSLOTS], [rows, cc]) * cev[c]
            dot = jnp.sum(acc)
            dv = jnp.full((LANES,), dot, jnp.float32)
            if p < LANES:
                z0 = jnp.where(lane == p, dv, z0)
            else:
                z1 = jnp.where(lane == (p - LANES), dv, z1)
            return z0, z1

        z0 = jnp.zeros((LANES,), jnp.float32)
        z1 = jnp.zeros((LANES,), jnp.float32)
        ha = [fire(p) for p in range(NSLOTS)]
        for h in ha:
            h.wait()
        for p in range(NSLOTS):
            z0, z1 = compute(p, z0, z1)
        hb = [fire(p) for p in range(NSLOTS, PATH_LEN)]
        for h in hb:
            h.wait()
        for p in range(NSLOTS, PATH_LEN):
            z0, z1 = compute(p, z0, z1)

        z0 = z0 * bm_v[pl.ds(0, LANES)]
        z1 = z1 * bm_v[pl.ds(LANES, LANES)]
        p0 = 1.0 / (1.0 + jnp.exp(-z0))
        p1 = 1.0 / (1.0 + jnp.exp(-z1))
        p1 = jnp.where(lane < (PATH_LEN - LANES), p1, jnp.float32(1.0))
        pv = p0 * p1

        r = pv[0]
        for l in range(1, LANES):
            r = r * pv[l]
        out_v[...] = jnp.full((LANES,), r, jnp.float32)
        pltpu.sync_copy(out_v, out_hbm)


@jax.jit
def _run(ce, idx, bm, matT):
    mesh = plsc.VectorSubcoreMesh(
        core_axis_name="c", subcore_axis_name="s", num_cores=1
    )
    f = pl.kernel(
        _sc_body,
        out_type=jax.ShapeDtypeStruct((LANES,), jnp.float32),
        mesh=mesh,
        compiler_params=pltpu.CompilerParams(needs_layout_passes=False),
        scratch_types=[
            pltpu.VMEM((2 * LANES,), jnp.int32),
            pltpu.VMEM((EMBED_SIZE,), jnp.float32),
            pltpu.VMEM((2 * LANES,), jnp.float32),
            pltpu.VMEM((NSLOTS, EMBED_SIZE, BLK), jnp.float32),
            pltpu.VMEM((LANES,), jnp.float32),
            pltpu.SemaphoreType.DMA,
        ],
    )
    out = f(ce, idx, bm, matT)
    return out[0]


def kernel(context_embedding, input_path_idxs, binary_multiplier, matrix):
    ce = context_embedding.reshape(EMBED_SIZE)
    idx = input_path_idxs.astype(jnp.int32)
    bm = binary_multiplier.reshape(PATH_LEN)
    return _run(ce, idx, bm, matrix.T)
